# baseline (speedup 1.0000x reference)
import functools

import jax
import jax.numpy as jnp
from jax import lax
from jax.experimental import pallas as pl
from jax.experimental.pallas import tpu as pltpu
from jax.experimental.pallas import tpu_sc as plsc


def _sc_fused_kernel(n_ids: int, d_model: int):
    info = plsc.get_sparse_core_info()
    nc, ns, lanes = info.num_cores, info.num_subcores, info.num_lanes
    nw = nc * ns
    per_w = n_ids // nw
    assert n_ids % nw == 0 and per_w % lanes == 0
    gchunk = 128
    assert per_w % gchunk == 0

    ch = 8
    nchunk = per_w // ch
    assert per_w % ch == 0 and nchunk >= 8
    celems = ch * d_model
    half = celems // 2
    groups = d_model // lanes
    nbuf = 3
    nsteady = nchunk - nbuf - 4
    assert nsteady % nbuf == 0

    mesh = plsc.VectorSubcoreMesh(core_axis_name="c", subcore_axis_name="s")

    @functools.partial(
        pl.kernel,
        mesh=mesh,
        out_type=jax.ShapeDtypeStruct((n_ids * d_model,), jnp.float32),
        scratch_types=[
            pltpu.VMEM((per_w,), jnp.int32),
            pltpu.VMEM((per_w,), jnp.float32),
            pltpu.VMEM((per_w + lanes,), jnp.float32),
            pltpu.VMEM((celems,), jnp.float32),
            pltpu.VMEM((celems,), jnp.float32),
            pltpu.VMEM((celems,), jnp.float32),
            pltpu.VMEM((celems,), jnp.float32),
            pltpu.VMEM((celems,), jnp.float32),
            pltpu.VMEM((celems,), jnp.float32),
            pltpu.SemaphoreType.DMA,
            pltpu.SemaphoreType.DMA((nbuf,)),
            pltpu.SemaphoreType.DMA((nbuf,)),
        ],
    )
    def sc_fused(ids_hbm, table_hbm, hid_hbm, out_hbm,
                 idx_v, w_v, delta_v, in_t0, in_t1, in_t2,
                 out_t0, out_t1, out_t2, gsem, in_sem, out_sem):
        in_ts = [in_t0, in_t1, in_t2]
        out_ts = [out_t0, out_t1, out_t2]
        wid = lax.axis_index("s") * nc + lax.axis_index("c")
        row0 = wid * per_w
        ebase = row0 * d_model

        pltpu.sync_copy(ids_hbm.at[pl.ds(row0, per_w)], idx_v)
        gathers = [
            pltpu.async_copy(
                table_hbm.at[idx_v.at[pl.ds(g * gchunk, gchunk)]],
                w_v.at[pl.ds(g * gchunk, gchunk)],
                gsem,
            )
            for g in range(per_w // gchunk)
        ]
        for cop in gathers:
            cop.wait()
        for i in range(per_w // lanes):
            sl = pl.ds(i * lanes, lanes)
            v = w_v[sl]
            delta_v[sl] = v - v

        def in_copies(c, slot):
            return [
                pltpu.make_async_copy(
                    hid_hbm.at[pl.ds(ebase + c * celems + h * half, half)],
                    in_ts[slot].at[pl.ds(h * half, half)],
                    in_sem.at[slot],
                )
                for h in range(2)
            ]

        def out_copies(c, slot):
            return [
                pltpu.make_async_copy(
                    out_ts[slot].at[pl.ds(h * half, half)],
                    out_hbm.at[pl.ds(ebase + c * celems + h * half, half)],
                    out_sem.at[slot],
                )
                for h in range(2)
            ]

        def start(cps):
            for cp in cps:
                cp.start()

        def wait(cps):
            for cp in cps:
                cp.wait()

        def compute(c, slot):
            dvec = delta_v[pl.ds(c * ch, lanes)]
            for r in range(ch):
                s = dvec[r]
                base = r * d_model

                i_t, o_t = in_ts[slot], out_ts[slot]

                @plsc.parallel_loop(0, groups, unroll=16)
                def grp_body(g, base=base, s=s, i_t=i_t, o_t=o_t):
                    sl = pl.ds(base + g * lanes, lanes)
                    o_t[sl] = i_t[sl] + s

        def step(c, slot, first, prefetch):
            wait(in_copies(c, slot))
            if not first:
                wait(out_copies(c - nbuf, slot))
            compute(c, slot)
            start(out_copies(c, slot))
            if prefetch:
                start(in_copies(c + nbuf, slot))

        for k in range(nbuf):
            start(in_copies(k, k))
        for k in range(nbuf):
            step(k, k, True, True)

        def loop_body(t, _):
            c0 = nbuf + nbuf * t
            for k in range(nbuf):
                step(c0 + k, k, False, True)
            return 0

        lax.fori_loop(0, nsteady // nbuf, loop_body, 0)

        for c in range(nchunk - 4, nchunk):
            step(c, c % nbuf, False, c + nbuf < nchunk)
        for c in range(nchunk - nbuf, nchunk):
            wait(out_copies(c, c % nbuf))

    return sc_fused


@functools.lru_cache(maxsize=None)
def _build(n_ids, d_model):
    return _sc_fused_kernel(n_ids, d_model)


def kernel(witness_ids, hidden_states, witness_weight):
    batch, seq = witness_ids.shape
    seq_h, batch_h, d_model = hidden_states.shape
    sc_fused = _build(batch * seq, d_model)
    ids_sb = witness_ids.T.reshape(-1).astype(jnp.int32)
    table = witness_weight.reshape(-1)
    out = sc_fused(ids_sb, table, hidden_states.reshape(-1))
    return out.reshape(seq_h, batch_h, d_model)


# speedup vs baseline: 2.4613x; 2.4613x over previous
import functools

import jax
import jax.numpy as jnp
from jax import lax
from jax.experimental import pallas as pl
from jax.experimental.pallas import tpu as pltpu
from jax.experimental.pallas import tpu_sc as plsc


def _sc_delta_kernel(n_ids: int):
    info = plsc.get_sparse_core_info()
    nc, ns, lanes = info.num_cores, info.num_subcores, info.num_lanes
    nw = nc * ns
    per_w = n_ids // nw
    assert n_ids % nw == 0 and per_w % lanes == 0
    gchunk = 128
    assert per_w % gchunk == 0

    mesh = plsc.VectorSubcoreMesh(core_axis_name="c", subcore_axis_name="s")

    @functools.partial(
        pl.kernel,
        mesh=mesh,
        out_type=jax.ShapeDtypeStruct((n_ids,), jnp.float32),
        scratch_types=[
            pltpu.VMEM((per_w,), jnp.int32),
            pltpu.VMEM((per_w,), jnp.float32),
            pltpu.VMEM((per_w,), jnp.float32),
            pltpu.SemaphoreType.DMA,
        ],
    )
    def sc_delta(ids_hbm, table_hbm, out_hbm, idx_v, rows_v, delta_v, sem):
        wid = lax.axis_index("s") * nc + lax.axis_index("c")
        base = wid * per_w
        pltpu.sync_copy(ids_hbm.at[pl.ds(base, per_w)], idx_v)
        gathers = [
            pltpu.async_copy(
                table_hbm.at[idx_v.at[pl.ds(g * gchunk, gchunk)]],
                rows_v.at[pl.ds(g * gchunk, gchunk)],
                sem,
            )
            for g in range(per_w // gchunk)
        ]
        for cop in gathers:
            cop.wait()
        for i in range(per_w // lanes):
            sl = pl.ds(i * lanes, lanes)
            v = rows_v[sl]
            delta_v[sl] = v - v
        pltpu.sync_copy(delta_v, out_hbm.at[pl.ds(base, per_w)])

    return sc_delta


@functools.lru_cache(maxsize=None)
def _build(n_ids):
    return _sc_delta_kernel(n_ids)


def kernel(witness_ids, hidden_states, witness_weight):
    batch, seq = witness_ids.shape
    seq_h, batch_h, d_model = hidden_states.shape
    sc_delta = _build(batch * seq)
    ids_flat = witness_ids.reshape(-1).astype(jnp.int32)
    table = witness_weight.reshape(-1)
    delta = sc_delta(ids_flat, table)
    delta_sb1 = jnp.transpose(delta.reshape(batch, seq), (1, 0))[:, :, None]
    return hidden_states + delta_sb1
